# Initial kernel scaffold; baseline (speedup 1.0000x reference)
import jax
import jax.numpy as jnp
from jax.experimental import pallas as pl


def kernel(x, edge_index, norm, weight, bias):
    raise NotImplementedError("write your pallas kernel here")


# speedup vs baseline: 4.1633x; 4.1633x over previous
import functools

import jax
import jax.numpy as jnp
from jax import lax
from jax.experimental import pallas as pl
from jax.experimental.pallas import tpu as pltpu
from jax.experimental.pallas import tpu_sc as plsc

N_NODES = 10000
N_EDGES = 320000
IN_CH = 128
OUT_CH = 128

NC = 2
NS = 16
CHUNK = 128

EDGES_PER_TILE = -(-N_EDGES // (NC * NS * CHUNK)) * CHUNK
E_PAD = EDGES_PER_TILE * NC * NS
CHUNKS_PER_TILE = EDGES_PER_TILE // CHUNK

N_ROWS = 10240
N_DUMMY = N_ROWS - N_NODES
ROWS_PER_TILE = N_ROWS // NS


def _mm_body(x_ref, nrm_ref, w_ref, o_ref):
    o_ref[...] = jnp.dot(
        x_ref[...] * nrm_ref[...], w_ref[...],
        preferred_element_type=jnp.float32,
    )


def _finish_body(acc_ref, nrm_ref, b_ref, o_ref):
    o_ref[...] = (acc_ref[0] + acc_ref[1]) * nrm_ref[...] + b_ref[...]


def _scatter_body(h_hbm, src_hbm, dst_hbm, zeros_hbm, out_hbm,
                  acc, idx_s, idx_d, msgs, sem):
    c = lax.axis_index("c")
    s = lax.axis_index("s")
    w = c * NS + s

    r0 = s * ROWS_PER_TILE
    pltpu.sync_copy(zeros_hbm.at[pl.ds(r0, ROWS_PER_TILE)],
                    acc.at[pl.ds(r0, ROWS_PER_TILE)])
    plsc.subcore_barrier()

    e0 = w * EDGES_PER_TILE

    def body(ci, carry):
        base = e0 + ci * CHUNK
        pltpu.sync_copy(src_hbm.at[pl.ds(base, CHUNK)], idx_s)
        pltpu.sync_copy(dst_hbm.at[pl.ds(base, CHUNK)], idx_d)
        pltpu.async_copy(h_hbm.at[idx_s], msgs, sem).wait()
        pltpu.sync_copy(msgs, acc.at[idx_d], add=True)
        return carry

    lax.fori_loop(0, CHUNKS_PER_TILE, body, 0)
    plsc.subcore_barrier()

    pltpu.sync_copy(acc.at[pl.ds(r0, ROWS_PER_TILE)],
                    out_hbm.at[c, pl.ds(r0, ROWS_PER_TILE)])


@jax.jit
def kernel(x, edge_index, norm, weight, bias):
    x = x.astype(jnp.float32)
    norm = norm.astype(jnp.float32)
    normb = jnp.broadcast_to(norm, (N_NODES, OUT_CH))

    src = edge_index[0].astype(jnp.int32)
    dst = edge_index[1].astype(jnp.int32)
    npad = E_PAD - N_EDGES
    src = jnp.concatenate([src, jnp.zeros((npad,), jnp.int32)])
    dst = jnp.concatenate(
        [dst, N_NODES + (jnp.arange(npad, dtype=jnp.int32) % N_DUMMY)])

    R = 1000
    h = pl.pallas_call(
        _mm_body,
        grid=(N_NODES // R,),
        in_specs=[
            pl.BlockSpec((R, IN_CH), lambda i: (i, 0)),
            pl.BlockSpec((R, IN_CH), lambda i: (i, 0)),
            pl.BlockSpec((IN_CH, OUT_CH), lambda i: (0, 0)),
        ],
        out_specs=pl.BlockSpec((R, OUT_CH), lambda i: (i, 0)),
        out_shape=jax.ShapeDtypeStruct((N_NODES, OUT_CH), jnp.float32),
    )(x, normb, weight)

    zeros = jnp.zeros((N_ROWS, OUT_CH), jnp.float32)
    mesh = plsc.VectorSubcoreMesh(core_axis_name="c", subcore_axis_name="s")
    acc2 = pl.kernel(
        _scatter_body,
        out_type=jax.ShapeDtypeStruct((NC, N_ROWS, OUT_CH), jnp.float32),
        mesh=mesh,
        scratch_types=[
            pltpu.VMEM_SHARED((N_ROWS, OUT_CH), jnp.float32),
            pltpu.VMEM((CHUNK,), jnp.int32),
            pltpu.VMEM((CHUNK,), jnp.int32),
            pltpu.VMEM((CHUNK, OUT_CH), jnp.float32),
            pltpu.SemaphoreType.DMA,
        ],
    )(h, src, dst, zeros)

    out = pl.pallas_call(
        _finish_body,
        grid=(N_NODES // R,),
        in_specs=[
            pl.BlockSpec((NC, R, OUT_CH), lambda i: (0, i, 0)),
            pl.BlockSpec((R, OUT_CH), lambda i: (i, 0)),
            pl.BlockSpec((1, OUT_CH), lambda i: (0, 0)),
        ],
        out_specs=pl.BlockSpec((R, OUT_CH), lambda i: (i, 0)),
        out_shape=jax.ShapeDtypeStruct((N_NODES, OUT_CH), jnp.float32),
    )(acc2, normb, bias.reshape(1, OUT_CH))
    return out
